# baseline (speedup 1.0000x reference)
import functools

import jax
import jax.numpy as jnp
from jax import lax
from jax.experimental import pallas as pl
from jax.experimental.pallas import tpu as pltpu
from jax.experimental.pallas import tpu_sc as plsc

F32 = jnp.float32
I32 = jnp.int32

D0, H0, W0 = 41, 128, 128
DP, HP, WP = 43, 130, 130
NPAD0 = DP * HP * WP
NSITES = 20000
NS = 20480
ZID = NS
NG2 = 43 * 4 * 65 * 65

_NW = 32
_SPW = NS // _NW


@functools.lru_cache(maxsize=None)
def _sc_mesh():
    return plsc.VectorSubcoreMesh(core_axis_name="c", subcore_axis_name="s")


_CHS = 128


@functools.lru_cache(maxsize=None)
def _make_sc_gather(cw, name):
    def body(table, idx, out, idx_v, rows_v, sem):
        wid = lax.axis_index("s") * 2 + lax.axis_index("c")
        base = wid * _SPW

        def step(c, carry):
            s = (base + c * _CHS) * 27
            pltpu.sync_copy(idx.at[pl.ds(s, _CHS * 27)], idx_v)
            pltpu.async_copy(table.at[idx_v], rows_v, sem).wait()
            pltpu.sync_copy(rows_v, out.at[pl.ds(s, _CHS * 27)])
            return carry

        lax.fori_loop(0, _SPW // _CHS, step, 0)

    return pl.kernel(
        body,
        out_type=jax.ShapeDtypeStruct((NS * 27, cw), F32),
        mesh=_sc_mesh(),
        scratch_types=[
            pltpu.VMEM((_CHS * 27,), I32),
            pltpu.VMEM((_CHS * 27, cw), F32),
            pltpu.SemaphoreType.DMA,
        ],
        compiler_params=pltpu.CompilerParams(use_tc_tiling_on_sc=False),
        name=name,
    )


def _sc_scatter_body(rows_hbm, idx_hbm, grid_ref, idx_v, rows_v, sem):
    wid = lax.axis_index("s") * 2 + lax.axis_index("c")
    base = wid * _SPW
    pltpu.sync_copy(idx_hbm.at[pl.ds(base, _SPW)], idx_v)
    pltpu.sync_copy(rows_hbm.at[pl.ds(base, _SPW)], rows_v)
    pltpu.async_copy(rows_v, grid_ref.at[idx_v], sem).wait()


@functools.lru_cache(maxsize=None)
def _sc_scatter16():
    return pl.kernel(
        _sc_scatter_body,
        out_type=(),
        mesh=_sc_mesh(),
        scratch_types=[
            pltpu.VMEM((_SPW,), I32),
            pltpu.VMEM((_SPW, 16), F32),
            pltpu.SemaphoreType.DMA,
        ],
        compiler_params=pltpu.CompilerParams(use_tc_tiling_on_sc=False),
        name="sc_scatter16",
    )


_DBG_JNP_GATHER = False
_DBG_JNP_SCATTER = False


def _gather27(table, idx27, cw, name):
    if _DBG_JNP_GATHER:
        return table[idx27.reshape(-1)].reshape(NS, 27 * cw)
    g = _make_sc_gather(cw, name)(table, idx27.reshape(NS * 27))
    return g.reshape(NS, 27 * cw)


def _scatter16(rows, idx, grid_rows):
    if _DBG_JNP_SCATTER:
        return jnp.zeros((grid_rows, 16), F32).at[idx].set(rows)
    gref = jax.new_ref(jnp.zeros((grid_rows, 16), F32))
    _sc_scatter16()(rows, idx, gref)
    return gref[...]


_BS1 = 2048


def _tc_l0_body(g_ref, w_ref, b_ref, y_ref):
    acc = jnp.dot(g_ref[...], w_ref[...], preferred_element_type=F32,
                  precision=lax.Precision.HIGHEST)
    y_ref[...] = jnp.maximum(acc + b_ref[...], 0.0)


def _tc_l0(g0, w0aug, b0):
    nblk = NS // _BS1
    return pl.pallas_call(
        _tc_l0_body,
        grid=(nblk,),
        in_specs=[
            pl.BlockSpec((_BS1, 216), lambda j: (j, 0)),
            pl.BlockSpec((216, 16), lambda j: (0, 0)),
            pl.BlockSpec((1, 16), lambda j: (0, 0)),
        ],
        out_specs=pl.BlockSpec((_BS1, 16), lambda j: (j, 0)),
        out_shape=jax.ShapeDtypeStruct((NS, 16), F32),
        name="tc_stage1_l0",
    )(g0, w0aug, b0)


def _tc_l1_body(g_ref, w_ref, b_ref, y_ref):
    acc = jnp.dot(g_ref[...], w_ref[...], preferred_element_type=F32,
                  precision=lax.Precision.HIGHEST)
    y_ref[...] = jnp.maximum(acc + b_ref[...], 0.0)


def _tc_l1(g1, w1, b1):
    nblk = NS // _BS1
    return pl.pallas_call(
        _tc_l1_body,
        grid=(nblk,),
        in_specs=[
            pl.BlockSpec((_BS1, 432), lambda j: (j, 0)),
            pl.BlockSpec((432, 16), lambda j: (0, 0)),
            pl.BlockSpec((1, 16), lambda j: (0, 0)),
        ],
        out_specs=pl.BlockSpec((_BS1, 16), lambda j: (j, 0)),
        out_shape=jax.ShapeDtypeStruct((NS, 16), F32),
        name="tc_stage1_l1",
    )(g1, w1, b1)


def _make_down(Do, Ho, Wo, HQ, WQ, C, Co, name):
    HW = Ho * Wo
    CL = 4 * C + 4

    def body(x0, x1, x2, w_ref, b_ref, y_ref, nm_ref, col_ref):
        xs = (x0, x1, x2)
        msum = jnp.zeros((Ho, Wo, 1), F32)
        for dz in range(3):
            for dy in range(3):
                for dx in range(3):
                    k = (dz * 3 + dy) * 3 + dx
                    p = (dy & 1) * 2 + (dx & 1)
                    hs, ws = dy >> 1, dx >> 1
                    col_ref[:, :, k * C:(k + 1) * C] = (
                        xs[dz][0, hs:hs + Ho, ws:ws + Wo, p * C:(p + 1) * C])
                    msum = msum + xs[dz][0, hs:hs + Ho, ws:ws + Wo,
                                         4 * C + p:4 * C + p + 1]
        colr = col_ref[...].reshape(HW, 27 * C)
        nm = (msum > 0).astype(F32).reshape(HW, 1)
        ch = min(HW, 512)
        for i in range(HW // ch):
            lo, hi = i * ch, (i + 1) * ch
            acc = jnp.dot(colr[lo:hi], w_ref[...], preferred_element_type=F32,
                          precision=lax.Precision.HIGHEST)
            y_ref[0, lo:hi] = jnp.maximum(acc + b_ref[...], 0.0) * nm[lo:hi]
        nm_ref[0] = nm

    x_specs = [pl.BlockSpec((1, HQ, WQ, CL),
                            (lambda d, dz=dz: (2 * d + dz, 0, 0, 0)))
               for dz in range(3)]
    return pl.pallas_call(
        body,
        grid=(Do,),
        in_specs=x_specs + [
            pl.BlockSpec((27 * C, Co), lambda d: (0, 0)),
            pl.BlockSpec((1, Co), lambda d: (0, 0)),
        ],
        out_specs=[
            pl.BlockSpec((1, HW, Co), lambda d: (d, 0, 0)),
            pl.BlockSpec((1, HW, 1), lambda d: (d, 0, 0)),
        ],
        out_shape=[
            jax.ShapeDtypeStruct((Do, HW, Co), F32),
            jax.ShapeDtypeStruct((Do, HW, 1), F32),
        ],
        scratch_shapes=[pltpu.VMEM((Ho, Wo, 27 * C), F32)],
        compiler_params=pltpu.CompilerParams(
            vmem_limit_bytes=100 * 1024 * 1024),
        name=name,
    )


def _make_subm(Do, Ho, Wo, C, Co, name):
    HW = Ho * Wo

    def body(x0, x1, x2, m_ref, w_ref, b_ref, y_ref, col_ref):
        xs = (x0, x1, x2)
        for dz in range(3):
            for dy in range(3):
                for dx in range(3):
                    k = (dz * 3 + dy) * 3 + dx
                    col_ref[:, :, k * C:(k + 1) * C] = (
                        xs[dz][0, dy:dy + Ho, dx:dx + Wo, :])
        colr = col_ref[...].reshape(HW, 27 * C)
        ch = min(HW, 512)
        for i in range(HW // ch):
            lo, hi = i * ch, (i + 1) * ch
            acc = jnp.dot(colr[lo:hi], w_ref[...], preferred_element_type=F32,
                          precision=lax.Precision.HIGHEST)
            y_ref[0, lo:hi] = (jnp.maximum(acc + b_ref[...], 0.0)
                               * m_ref[0, lo:hi])

    x_specs = [pl.BlockSpec((1, Ho + 2, Wo + 2, C),
                            (lambda d, dz=dz: (d + dz, 0, 0, 0)))
               for dz in range(3)]
    return pl.pallas_call(
        body,
        grid=(Do,),
        in_specs=x_specs + [
            pl.BlockSpec((1, HW, 1), lambda d: (d, 0, 0)),
            pl.BlockSpec((27 * C, Co), lambda d: (0, 0)),
            pl.BlockSpec((1, Co), lambda d: (0, 0)),
        ],
        out_specs=pl.BlockSpec((1, HW, Co), lambda d: (d, 0, 0)),
        out_shape=jax.ShapeDtypeStruct((Do, HW, Co), F32),
        scratch_shapes=[pltpu.VMEM((Ho, Wo, 27 * C), F32)],
        compiler_params=pltpu.CompilerParams(
            vmem_limit_bytes=100 * 1024 * 1024),
        name=name,
    )


def _co_body(x0, x1, x2, m0, m1, m2, w_ref, b_ref, o_ref, col_ref):
    xs = (x0, x1, x2)
    ms = (m0, m1, m2)
    msum = jnp.zeros((256, 1), F32)
    for dz in range(3):
        col_ref[:, dz * 64:(dz + 1) * 64] = xs[dz][0]
        msum = msum + ms[dz][0]
    acc = jnp.dot(col_ref[...], w_ref[...], preferred_element_type=F32,
                  precision=lax.Precision.HIGHEST)
    nm = (msum > 0).astype(F32)
    o_ref[0] = jnp.maximum(acc + b_ref[...], 0.0) * nm


def _conv_out(x, m, w, b):
    x_specs = [pl.BlockSpec((1, 256, 64), (lambda d, dz=dz: (2 * d + dz, 0, 0)))
               for dz in range(3)]
    m_specs = [pl.BlockSpec((1, 256, 1), (lambda d, dz=dz: (2 * d + dz, 0, 0)))
               for dz in range(3)]
    return pl.pallas_call(
        _co_body,
        grid=(2,),
        in_specs=x_specs + m_specs + [
            pl.BlockSpec((192, 128), lambda d: (0, 0)),
            pl.BlockSpec((1, 128), lambda d: (0, 0)),
        ],
        out_specs=pl.BlockSpec((1, 256, 128), lambda d: (d, 0, 0)),
        out_shape=jax.ShapeDtypeStruct((2, 256, 128), F32),
        scratch_shapes=[pltpu.VMEM((256, 192), F32)],
        name="tc_conv_out",
    )(x, x, x, m, m, m, w, b)


def _fold_w(params, i):
    w = params['w%d' % i]
    scale = params['g%d' % i] / jnp.sqrt(1.0 + 1e-3)
    co, ci = w.shape[0], w.shape[1]
    wr = jnp.transpose(w, (2, 3, 4, 1, 0)).reshape(-1, co) * scale[None, :]
    return wr.astype(F32), params['b%d' % i].reshape(1, co).astype(F32)


def _parity(xp):
    d, h2, w2, c = xp.shape
    return (xp.reshape(d, h2 // 2, 2, w2 // 2, 2, c)
            .transpose(0, 1, 3, 2, 4, 5)
            .reshape(d, h2 // 2, w2 // 2, 4 * c))


def _pad3(x):
    return jnp.pad(x, ((1, 1), (1, 1), (1, 1), (0, 0)))


def kernel(voxel_features, voxel_coords, params):
    z = voxel_coords[:, 1] + 1
    y = voxel_coords[:, 2] + 1
    x = voxel_coords[:, 3] + 1

    ids = jnp.arange(NSITES, dtype=F32)[:, None]
    val8 = jnp.concatenate(
        [voxel_features.astype(F32), ids, jnp.zeros((NSITES, 3), F32)], axis=1)
    tab = (jnp.zeros((DP, HP, WP, 8), F32).at[:, :, :, 4].set(float(ZID))
           .at[z, y, x].set(val8)
           .reshape(NPAD0, 8))

    f0 = ((z * HP + y) * WP + x).astype(I32)
    safe = jnp.full((NS - NSITES,), (1 * HP + 1) * WP + 1, I32)
    f0p = jnp.concatenate([f0, safe])
    offs = jnp.array([dz * HP * WP + dy * WP + dx
                      for dz in (-1, 0, 1) for dy in (-1, 0, 1)
                      for dx in (-1, 0, 1)], I32)
    idx27 = f0p[:, None] + offs[None, :]

    w0, b0 = _fold_w(params, 0)
    w0aug = jnp.zeros((27, 8, 16), F32).at[:, :4, :].set(
        w0.reshape(27, 4, 16)).reshape(216, 16)

    g0 = _gather27(tab, idx27, 8, "sc_gather_l0")
    y0 = _tc_l0(g0, w0aug, b0)
    nbr = lax.slice(g0, (0, 4), (NS, 216), (1, 8)).astype(I32)
    y0t = jnp.concatenate([y0, jnp.zeros((1, 16), F32)], axis=0)

    w1, b1 = _fold_w(params, 1)
    g1 = _gather27(y0t, nbr, 16, "sc_gather_l1")
    y1 = _tc_l1(g1, w1, b1)

    p = (y & 1) * 2 + (x & 1)
    f2 = ((((z * 65 + (y >> 1)) * 65 + (x >> 1)) * 4) + p).astype(I32)
    f2p = jnp.concatenate([f2, jnp.full((NS - NSITES,), NG2, I32)])
    grid2 = _scatter16(y1, f2p, NG2 + 8)[:NG2].reshape(43, 65, 65, 64)

    md = jnp.zeros((DP, HP, WP, 1), F32).at[z, y, x, 0].set(1.0)
    xm2 = jnp.concatenate([grid2, _parity(md)], axis=-1)

    w2, b2 = _fold_w(params, 2)
    y2, nm2 = _make_down(21, 64, 64, 65, 65, 16, 32, "tc_conv2")(
        xm2, xm2, xm2, w2, b2)

    w3, b3 = _fold_w(params, 3)
    x3 = _pad3(y2.reshape(21, 64, 64, 32))
    y3 = _make_subm(21, 64, 64, 32, 32, "tc_subm3")(x3, x3, x3, nm2, w3, b3)

    w4, b4 = _fold_w(params, 4)
    x4 = _pad3(y3.reshape(21, 64, 64, 32))
    y4 = _make_subm(21, 64, 64, 32, 32, "tc_subm4")(x4, x4, x4, nm2, w4, b4)

    w5, b5 = _fold_w(params, 5)
    xm3 = jnp.concatenate([_parity(_pad3(y4.reshape(21, 64, 64, 32))),
                           _parity(_pad3(nm2.reshape(21, 64, 64, 1)))],
                          axis=-1)
    y5, nm3 = _make_down(11, 32, 32, 33, 33, 32, 64, "tc_conv3")(
        xm3, xm3, xm3, w5, b5)

    w6, b6 = _fold_w(params, 6)
    x6 = _pad3(y5.reshape(11, 32, 32, 64))
    y6 = _make_subm(11, 32, 32, 64, 64, "tc_subm6")(x6, x6, x6, nm3, w6, b6)

    w7, b7 = _fold_w(params, 7)
    x7 = _pad3(y6.reshape(11, 32, 32, 64))
    y7 = _make_subm(11, 32, 32, 64, 64, "tc_subm7")(x7, x7, x7, nm3, w7, b7)

    w8, b8 = _fold_w(params, 8)
    padhw = ((0, 0), (1, 1), (1, 1), (0, 0))
    xm4 = jnp.concatenate(
        [_parity(jnp.pad(y7.reshape(11, 32, 32, 64), padhw)),
         _parity(jnp.pad(nm3.reshape(11, 32, 32, 1), padhw))],
        axis=-1)
    y8, nm4 = _make_down(5, 16, 16, 17, 17, 64, 64, "tc_conv4")(
        xm4, xm4, xm4, w8, b8)

    w9, b9 = _fold_w(params, 9)
    x9 = _pad3(y8.reshape(5, 16, 16, 64))
    y9 = _make_subm(5, 16, 16, 64, 64, "tc_subm9")(x9, x9, x9, nm4, w9, b9)

    w10, b10 = _fold_w(params, 10)
    x10 = _pad3(y9.reshape(5, 16, 16, 64))
    y10 = _make_subm(5, 16, 16, 64, 64, "tc_subm10")(x10, x10, x10, nm4,
                                                     w10, b10)

    w11, b11 = _fold_w(params, 11)
    out = _conv_out(y10, nm4, w11, b11)

    return out.reshape(2, 16, 16, 128).transpose(3, 0, 1, 2)[None]


# speedup vs baseline: 1.3724x; 1.3724x over previous
import functools

import jax
import jax.numpy as jnp
from jax import lax
from jax.experimental import pallas as pl
from jax.experimental.pallas import tpu as pltpu
from jax.experimental.pallas import tpu_sc as plsc

F32 = jnp.float32
I32 = jnp.int32

D0, H0, W0 = 41, 128, 128
DP, HP, WP = 43, 130, 130
NPAD0 = DP * HP * WP
NSITES = 20000
NS = 20480
ZPOOL = 8192
NG2 = 43 * 4 * 65 * 65

_NW = 32
_SPW = NS // _NW


@functools.lru_cache(maxsize=None)
def _sc_mesh():
    return plsc.VectorSubcoreMesh(core_axis_name="c", subcore_axis_name="s")


_CHS = 128


@functools.lru_cache(maxsize=None)
def _make_sc_gather(cw, name):
    def body(table, idx, out, idx_v, rows_v, sem):
        wid = lax.axis_index("s") * 2 + lax.axis_index("c")
        base = wid * _SPW

        def step(c, carry):
            s = (base + c * _CHS) * 27
            pltpu.sync_copy(idx.at[pl.ds(s, _CHS * 27)], idx_v)
            pltpu.async_copy(table.at[idx_v], rows_v, sem).wait()
            pltpu.sync_copy(rows_v, out.at[pl.ds(s, _CHS * 27)])
            return carry

        lax.fori_loop(0, _SPW // _CHS, step, 0)

    return pl.kernel(
        body,
        out_type=jax.ShapeDtypeStruct((NS * 27, cw), F32),
        mesh=_sc_mesh(),
        scratch_types=[
            pltpu.VMEM((_CHS * 27,), I32),
            pltpu.VMEM((_CHS * 27, cw), F32),
            pltpu.SemaphoreType.DMA,
        ],
        compiler_params=pltpu.CompilerParams(use_tc_tiling_on_sc=False),
        name=name,
    )


def _sc_scatter_body(rows_hbm, idx_hbm, grid_ref, idx_v, rows_v, sem):
    wid = lax.axis_index("s") * 2 + lax.axis_index("c")
    base = wid * _SPW
    pltpu.sync_copy(idx_hbm.at[pl.ds(base, _SPW)], idx_v)
    pltpu.sync_copy(rows_hbm.at[pl.ds(base, _SPW)], rows_v)
    pltpu.async_copy(rows_v, grid_ref.at[idx_v], sem).wait()


@functools.lru_cache(maxsize=None)
def _sc_scatter16():
    return pl.kernel(
        _sc_scatter_body,
        out_type=(),
        mesh=_sc_mesh(),
        scratch_types=[
            pltpu.VMEM((_SPW,), I32),
            pltpu.VMEM((_SPW, 16), F32),
            pltpu.SemaphoreType.DMA,
        ],
        compiler_params=pltpu.CompilerParams(use_tc_tiling_on_sc=False),
        name="sc_scatter16",
    )


_DBG_JNP_GATHER = False
_DBG_JNP_SCATTER = False


def _gather27(table, idx27, cw, name):
    if _DBG_JNP_GATHER:
        return table[idx27.reshape(-1)].reshape(NS, 27 * cw)
    g = _make_sc_gather(cw, name)(table, idx27.reshape(NS * 27))
    return g.reshape(NS, 27 * cw)


def _scatter16(rows, idx, grid_rows):
    if _DBG_JNP_SCATTER:
        return jnp.zeros((grid_rows, 16), F32).at[idx].set(rows)
    gref = jax.new_ref(jnp.zeros((grid_rows, 16), F32))
    _sc_scatter16()(rows, idx, gref)
    return gref[...]


_BS1 = 2048


def _tc_l0_body(g_ref, w_ref, b_ref, y_ref):
    acc = jnp.dot(g_ref[...], w_ref[...], preferred_element_type=F32,
                  precision=lax.Precision.HIGHEST)
    y_ref[...] = jnp.maximum(acc + b_ref[...], 0.0)


def _tc_l0(g0, w0aug, b0):
    nblk = NS // _BS1
    return pl.pallas_call(
        _tc_l0_body,
        grid=(nblk,),
        in_specs=[
            pl.BlockSpec((_BS1, 216), lambda j: (j, 0)),
            pl.BlockSpec((216, 16), lambda j: (0, 0)),
            pl.BlockSpec((1, 16), lambda j: (0, 0)),
        ],
        out_specs=pl.BlockSpec((_BS1, 16), lambda j: (j, 0)),
        out_shape=jax.ShapeDtypeStruct((NS, 16), F32),
        name="tc_stage1_l0",
    )(g0, w0aug, b0)


def _tc_l1_body(g_ref, w_ref, b_ref, y_ref):
    acc = jnp.dot(g_ref[...], w_ref[...], preferred_element_type=F32,
                  precision=lax.Precision.HIGHEST)
    y_ref[...] = jnp.maximum(acc + b_ref[...], 0.0)


def _tc_l1(g1, w1, b1):
    nblk = NS // _BS1
    return pl.pallas_call(
        _tc_l1_body,
        grid=(nblk,),
        in_specs=[
            pl.BlockSpec((_BS1, 432), lambda j: (j, 0)),
            pl.BlockSpec((432, 16), lambda j: (0, 0)),
            pl.BlockSpec((1, 16), lambda j: (0, 0)),
        ],
        out_specs=pl.BlockSpec((_BS1, 16), lambda j: (j, 0)),
        out_shape=jax.ShapeDtypeStruct((NS, 16), F32),
        name="tc_stage1_l1",
    )(g1, w1, b1)


def _make_down(Do, Ho, Wo, HQ, WQ, C, Co, name):
    HW = Ho * Wo
    CL = 4 * C + 4

    def body(x0, x1, x2, w_ref, b_ref, y_ref, nm_ref, col_ref):
        xs = (x0, x1, x2)
        msum = jnp.zeros((Ho, Wo, 1), F32)
        for dz in range(3):
            for dy in range(3):
                for dx in range(3):
                    k = (dz * 3 + dy) * 3 + dx
                    p = (dy & 1) * 2 + (dx & 1)
                    hs, ws = dy >> 1, dx >> 1
                    col_ref[:, :, k * C:(k + 1) * C] = (
                        xs[dz][0, hs:hs + Ho, ws:ws + Wo, p * C:(p + 1) * C])
                    msum = msum + xs[dz][0, hs:hs + Ho, ws:ws + Wo,
                                         4 * C + p:4 * C + p + 1]
        colr = col_ref[...].reshape(HW, 27 * C)
        nm = (msum > 0).astype(F32).reshape(HW, 1)
        ch = min(HW, 512)
        for i in range(HW // ch):
            lo, hi = i * ch, (i + 1) * ch
            acc = jnp.dot(colr[lo:hi], w_ref[...], preferred_element_type=F32,
                          precision=lax.Precision.HIGHEST)
            y_ref[0, lo:hi] = jnp.maximum(acc + b_ref[...], 0.0) * nm[lo:hi]
        nm_ref[0] = nm

    x_specs = [pl.BlockSpec((1, HQ, WQ, CL),
                            (lambda d, dz=dz: (2 * d + dz, 0, 0, 0)))
               for dz in range(3)]
    return pl.pallas_call(
        body,
        grid=(Do,),
        in_specs=x_specs + [
            pl.BlockSpec((27 * C, Co), lambda d: (0, 0)),
            pl.BlockSpec((1, Co), lambda d: (0, 0)),
        ],
        out_specs=[
            pl.BlockSpec((1, HW, Co), lambda d: (d, 0, 0)),
            pl.BlockSpec((1, HW, 1), lambda d: (d, 0, 0)),
        ],
        out_shape=[
            jax.ShapeDtypeStruct((Do, HW, Co), F32),
            jax.ShapeDtypeStruct((Do, HW, 1), F32),
        ],
        scratch_shapes=[pltpu.VMEM((Ho, Wo, 27 * C), F32)],
        compiler_params=pltpu.CompilerParams(
            vmem_limit_bytes=100 * 1024 * 1024),
        name=name,
    )


def _make_subm(Do, Ho, Wo, C, Co, name):
    HW = Ho * Wo

    def body(x0, x1, x2, m_ref, w_ref, b_ref, y_ref, col_ref):
        xs = (x0, x1, x2)
        for dz in range(3):
            for dy in range(3):
                for dx in range(3):
                    k = (dz * 3 + dy) * 3 + dx
                    col_ref[:, :, k * C:(k + 1) * C] = (
                        xs[dz][0, dy:dy + Ho, dx:dx + Wo, :])
        colr = col_ref[...].reshape(HW, 27 * C)
        ch = min(HW, 512)
        for i in range(HW // ch):
            lo, hi = i * ch, (i + 1) * ch
            acc = jnp.dot(colr[lo:hi], w_ref[...], preferred_element_type=F32,
                          precision=lax.Precision.HIGHEST)
            y_ref[0, lo:hi] = (jnp.maximum(acc + b_ref[...], 0.0)
                               * m_ref[0, lo:hi])

    x_specs = [pl.BlockSpec((1, Ho + 2, Wo + 2, C),
                            (lambda d, dz=dz: (d + dz, 0, 0, 0)))
               for dz in range(3)]
    return pl.pallas_call(
        body,
        grid=(Do,),
        in_specs=x_specs + [
            pl.BlockSpec((1, HW, 1), lambda d: (d, 0, 0)),
            pl.BlockSpec((27 * C, Co), lambda d: (0, 0)),
            pl.BlockSpec((1, Co), lambda d: (0, 0)),
        ],
        out_specs=pl.BlockSpec((1, HW, Co), lambda d: (d, 0, 0)),
        out_shape=jax.ShapeDtypeStruct((Do, HW, Co), F32),
        scratch_shapes=[pltpu.VMEM((Ho, Wo, 27 * C), F32)],
        compiler_params=pltpu.CompilerParams(
            vmem_limit_bytes=100 * 1024 * 1024),
        name=name,
    )


def _co_body(x0, x1, x2, m0, m1, m2, w_ref, b_ref, o_ref, col_ref):
    xs = (x0, x1, x2)
    ms = (m0, m1, m2)
    msum = jnp.zeros((256, 1), F32)
    for dz in range(3):
        col_ref[:, dz * 64:(dz + 1) * 64] = xs[dz][0]
        msum = msum + ms[dz][0]
    acc = jnp.dot(col_ref[...], w_ref[...], preferred_element_type=F32,
                  precision=lax.Precision.HIGHEST)
    nm = (msum > 0).astype(F32)
    o_ref[0] = jnp.maximum(acc + b_ref[...], 0.0) * nm


def _conv_out(x, m, w, b):
    x_specs = [pl.BlockSpec((1, 256, 64), (lambda d, dz=dz: (2 * d + dz, 0, 0)))
               for dz in range(3)]
    m_specs = [pl.BlockSpec((1, 256, 1), (lambda d, dz=dz: (2 * d + dz, 0, 0)))
               for dz in range(3)]
    return pl.pallas_call(
        _co_body,
        grid=(2,),
        in_specs=x_specs + m_specs + [
            pl.BlockSpec((192, 128), lambda d: (0, 0)),
            pl.BlockSpec((1, 128), lambda d: (0, 0)),
        ],
        out_specs=pl.BlockSpec((1, 256, 128), lambda d: (d, 0, 0)),
        out_shape=jax.ShapeDtypeStruct((2, 256, 128), F32),
        scratch_shapes=[pltpu.VMEM((256, 192), F32)],
        name="tc_conv_out",
    )(x, x, x, m, m, m, w, b)


def _fold_w(params, i):
    w = params['w%d' % i]
    scale = params['g%d' % i] / jnp.sqrt(1.0 + 1e-3)
    co, ci = w.shape[0], w.shape[1]
    wr = jnp.transpose(w, (2, 3, 4, 1, 0)).reshape(-1, co) * scale[None, :]
    return wr.astype(F32), params['b%d' % i].reshape(1, co).astype(F32)


def _parity(xp):
    d, h2, w2, c = xp.shape
    return (xp.reshape(d, h2 // 2, 2, w2 // 2, 2, c)
            .transpose(0, 1, 3, 2, 4, 5)
            .reshape(d, h2 // 2, w2 // 2, 4 * c))


def _pad3(x):
    return jnp.pad(x, ((1, 1), (1, 1), (1, 1), (0, 0)))


def kernel(voxel_features, voxel_coords, params):
    z = voxel_coords[:, 1] + 1
    y = voxel_coords[:, 2] + 1
    x = voxel_coords[:, 3] + 1

    ids = jnp.arange(NSITES, dtype=F32)[:, None]
    val8 = jnp.concatenate(
        [voxel_features.astype(F32), ids, jnp.zeros((NSITES, 3), F32)], axis=1)
    zid = (NS + jnp.arange(NPAD0, dtype=I32) % ZPOOL).astype(F32)
    tab = (jnp.zeros((NPAD0, 8), F32).at[:, 4].set(zid)
           .reshape(DP, HP, WP, 8)
           .at[z, y, x].set(val8)
           .reshape(NPAD0, 8))

    f0 = ((z * HP + y) * WP + x).astype(I32)
    safe = jnp.full((NS - NSITES,), (1 * HP + 1) * WP + 1, I32)
    f0p = jnp.concatenate([f0, safe])
    offs = jnp.array([dz * HP * WP + dy * WP + dx
                      for dz in (-1, 0, 1) for dy in (-1, 0, 1)
                      for dx in (-1, 0, 1)], I32)
    idx27 = f0p[:, None] + offs[None, :]

    w0, b0 = _fold_w(params, 0)
    w0aug = jnp.zeros((27, 8, 16), F32).at[:, :4, :].set(
        w0.reshape(27, 4, 16)).reshape(216, 16)

    g0 = _gather27(tab, idx27, 8, "sc_gather_l0")
    y0 = _tc_l0(g0, w0aug, b0)
    nbr = lax.slice(g0, (0, 4), (NS, 216), (1, 8)).astype(I32)
    y0t = jnp.concatenate([y0, jnp.zeros((ZPOOL, 16), F32)], axis=0)

    w1, b1 = _fold_w(params, 1)
    g1 = _gather27(y0t, nbr, 16, "sc_gather_l1")
    y1 = _tc_l1(g1, w1, b1)

    p = (y & 1) * 2 + (x & 1)
    f2 = ((((z * 65 + (y >> 1)) * 65 + (x >> 1)) * 4) + p).astype(I32)
    f2p = jnp.concatenate([f2, jnp.full((NS - NSITES,), NG2, I32)])
    grid2 = _scatter16(y1, f2p, NG2 + 8)[:NG2].reshape(43, 65, 65, 64)

    md = jnp.zeros((DP, HP, WP, 1), F32).at[z, y, x, 0].set(1.0)
    xm2 = jnp.concatenate([grid2, _parity(md)], axis=-1)

    w2, b2 = _fold_w(params, 2)
    y2, nm2 = _make_down(21, 64, 64, 65, 65, 16, 32, "tc_conv2")(
        xm2, xm2, xm2, w2, b2)

    w3, b3 = _fold_w(params, 3)
    x3 = _pad3(y2.reshape(21, 64, 64, 32))
    y3 = _make_subm(21, 64, 64, 32, 32, "tc_subm3")(x3, x3, x3, nm2, w3, b3)

    w4, b4 = _fold_w(params, 4)
    x4 = _pad3(y3.reshape(21, 64, 64, 32))
    y4 = _make_subm(21, 64, 64, 32, 32, "tc_subm4")(x4, x4, x4, nm2, w4, b4)

    w5, b5 = _fold_w(params, 5)
    xm3 = jnp.concatenate([_parity(_pad3(y4.reshape(21, 64, 64, 32))),
                           _parity(_pad3(nm2.reshape(21, 64, 64, 1)))],
                          axis=-1)
    y5, nm3 = _make_down(11, 32, 32, 33, 33, 32, 64, "tc_conv3")(
        xm3, xm3, xm3, w5, b5)

    w6, b6 = _fold_w(params, 6)
    x6 = _pad3(y5.reshape(11, 32, 32, 64))
    y6 = _make_subm(11, 32, 32, 64, 64, "tc_subm6")(x6, x6, x6, nm3, w6, b6)

    w7, b7 = _fold_w(params, 7)
    x7 = _pad3(y6.reshape(11, 32, 32, 64))
    y7 = _make_subm(11, 32, 32, 64, 64, "tc_subm7")(x7, x7, x7, nm3, w7, b7)

    w8, b8 = _fold_w(params, 8)
    padhw = ((0, 0), (1, 1), (1, 1), (0, 0))
    xm4 = jnp.concatenate(
        [_parity(jnp.pad(y7.reshape(11, 32, 32, 64), padhw)),
         _parity(jnp.pad(nm3.reshape(11, 32, 32, 1), padhw))],
        axis=-1)
    y8, nm4 = _make_down(5, 16, 16, 17, 17, 64, 64, "tc_conv4")(
        xm4, xm4, xm4, w8, b8)

    w9, b9 = _fold_w(params, 9)
    x9 = _pad3(y8.reshape(5, 16, 16, 64))
    y9 = _make_subm(5, 16, 16, 64, 64, "tc_subm9")(x9, x9, x9, nm4, w9, b9)

    w10, b10 = _fold_w(params, 10)
    x10 = _pad3(y9.reshape(5, 16, 16, 64))
    y10 = _make_subm(5, 16, 16, 64, 64, "tc_subm10")(x10, x10, x10, nm4,
                                                     w10, b10)

    w11, b11 = _fold_w(params, 11)
    out = _conv_out(y10, nm4, w11, b11)

    return out.reshape(2, 16, 16, 128).transpose(3, 0, 1, 2)[None]


# speedup vs baseline: 1.9442x; 1.4166x over previous
import functools

import jax
import jax.numpy as jnp
from jax import lax
from jax.experimental import pallas as pl
from jax.experimental.pallas import tpu as pltpu
from jax.experimental.pallas import tpu_sc as plsc

F32 = jnp.float32
I32 = jnp.int32

D0, H0, W0 = 41, 128, 128
DP, HP, WP = 43, 130, 130
NPAD0 = DP * HP * WP
NSITES = 20000
NS = 20480
ZPOOL = 8192
NG2 = 43 * 4 * 65 * 65

_NW = 32
_SPW = NS // _NW


@functools.lru_cache(maxsize=None)
def _sc_mesh():
    return plsc.VectorSubcoreMesh(core_axis_name="c", subcore_axis_name="s")


_CHS = 128


@functools.lru_cache(maxsize=None)
def _make_sc_gather(cw, name):
    def body(table, idx, out, idx_v, rows_v, sem):
        wid = lax.axis_index("s") * 2 + lax.axis_index("c")
        base = wid * _SPW

        def step(c, carry):
            s = (base + c * _CHS) * 27
            pltpu.sync_copy(idx.at[pl.ds(s, _CHS * 27)], idx_v)
            pltpu.async_copy(table.at[idx_v], rows_v, sem).wait()
            pltpu.sync_copy(rows_v, out.at[pl.ds(s, _CHS * 27)])
            return carry

        lax.fori_loop(0, _SPW // _CHS, step, 0)

    return pl.kernel(
        body,
        out_type=jax.ShapeDtypeStruct((NS * 27, cw), F32),
        mesh=_sc_mesh(),
        scratch_types=[
            pltpu.VMEM((_CHS * 27,), I32),
            pltpu.VMEM((_CHS * 27, cw), F32),
            pltpu.SemaphoreType.DMA,
        ],
        compiler_params=pltpu.CompilerParams(use_tc_tiling_on_sc=False),
        name=name,
    )


def _sc_scatter_body(rows_hbm, idx_hbm, grid_ref, idx_v, rows_v, sem):
    wid = lax.axis_index("s") * 2 + lax.axis_index("c")
    base = wid * _SPW
    pltpu.sync_copy(idx_hbm.at[pl.ds(base, _SPW)], idx_v)
    pltpu.sync_copy(rows_hbm.at[pl.ds(base, _SPW)], rows_v)
    pltpu.async_copy(rows_v, grid_ref.at[idx_v], sem).wait()


@functools.lru_cache(maxsize=None)
def _sc_scatter(cw):
    return pl.kernel(
        _sc_scatter_body,
        out_type=(),
        mesh=_sc_mesh(),
        scratch_types=[
            pltpu.VMEM((_SPW,), I32),
            pltpu.VMEM((_SPW, cw), F32),
            pltpu.SemaphoreType.DMA,
        ],
        compiler_params=pltpu.CompilerParams(use_tc_tiling_on_sc=False),
        name="sc_scatter%d" % cw,
    )


_DBG_JNP_GATHER = False
_DBG_JNP_SCATTER = False


def _gather27(table, idx27, cw, name):
    if _DBG_JNP_GATHER:
        return table[idx27.reshape(-1)].reshape(NS, 27 * cw)
    g = _make_sc_gather(cw, name)(table, idx27.reshape(NS * 27))
    return g.reshape(NS, 27 * cw)


def _scatter16(rows, idx, grid_rows):
    if _DBG_JNP_SCATTER:
        return jnp.zeros((grid_rows, 16), F32).at[idx].set(rows)
    gref = jax.new_ref(jnp.zeros((grid_rows, 16), F32))
    _sc_scatter(16)(rows, idx, gref)
    return gref[...]


_BS1 = 2048


def _tc_l0_body(g_ref, w_ref, b_ref, y_ref):
    acc = jnp.dot(g_ref[...], w_ref[...], preferred_element_type=F32,
                  precision=lax.Precision.HIGHEST)
    y_ref[...] = jnp.maximum(acc + b_ref[...], 0.0)


def _tc_l0(g0, w0aug, b0):
    nblk = NS // _BS1
    return pl.pallas_call(
        _tc_l0_body,
        grid=(nblk,),
        in_specs=[
            pl.BlockSpec((_BS1, 216), lambda j: (j, 0)),
            pl.BlockSpec((216, 16), lambda j: (0, 0)),
            pl.BlockSpec((1, 16), lambda j: (0, 0)),
        ],
        out_specs=pl.BlockSpec((_BS1, 16), lambda j: (j, 0)),
        out_shape=jax.ShapeDtypeStruct((NS, 16), F32),
        name="tc_stage1_l0",
    )(g0, w0aug, b0)


def _tc_l1_body(g_ref, w_ref, b_ref, y_ref):
    acc = jnp.dot(g_ref[...], w_ref[...], preferred_element_type=F32,
                  precision=lax.Precision.HIGHEST)
    y_ref[...] = jnp.maximum(acc + b_ref[...], 0.0)


def _tc_l1(g1, w1, b1):
    nblk = NS // _BS1
    return pl.pallas_call(
        _tc_l1_body,
        grid=(nblk,),
        in_specs=[
            pl.BlockSpec((_BS1, 432), lambda j: (j, 0)),
            pl.BlockSpec((432, 16), lambda j: (0, 0)),
            pl.BlockSpec((1, 16), lambda j: (0, 0)),
        ],
        out_specs=pl.BlockSpec((_BS1, 16), lambda j: (j, 0)),
        out_shape=jax.ShapeDtypeStruct((NS, 16), F32),
        name="tc_stage1_l1",
    )(g1, w1, b1)


def _make_down(Do, Ho, Wo, HQ, WQ, C, Co, name):
    HW = Ho * Wo
    CL = 4 * C + 4

    def body(x0, x1, x2, w_ref, b_ref, y_ref, nm_ref, col_ref):
        xs = (x0, x1, x2)
        msum = jnp.zeros((Ho, Wo, 1), F32)
        for dz in range(3):
            for dy in range(3):
                for dx in range(3):
                    k = (dz * 3 + dy) * 3 + dx
                    p = (dy & 1) * 2 + (dx & 1)
                    hs, ws = dy >> 1, dx >> 1
                    col_ref[:, :, k * C:(k + 1) * C] = (
                        xs[dz][0, hs:hs + Ho, ws:ws + Wo, p * C:(p + 1) * C])
                    msum = msum + xs[dz][0, hs:hs + Ho, ws:ws + Wo,
                                         4 * C + p:4 * C + p + 1]
        colr = col_ref[...].reshape(HW, 27 * C)
        nm = (msum > 0).astype(F32).reshape(HW, 1)
        ch = min(HW, 512)
        for i in range(HW // ch):
            lo, hi = i * ch, (i + 1) * ch
            acc = jnp.dot(colr[lo:hi], w_ref[...], preferred_element_type=F32,
                          precision=lax.Precision.HIGHEST)
            y_ref[0, lo:hi] = jnp.maximum(acc + b_ref[...], 0.0) * nm[lo:hi]
        nm_ref[0] = nm

    x_specs = [pl.BlockSpec((1, HQ, WQ, CL),
                            (lambda d, dz=dz: (2 * d + dz, 0, 0, 0)))
               for dz in range(3)]
    return pl.pallas_call(
        body,
        grid=(Do,),
        in_specs=x_specs + [
            pl.BlockSpec((27 * C, Co), lambda d: (0, 0)),
            pl.BlockSpec((1, Co), lambda d: (0, 0)),
        ],
        out_specs=[
            pl.BlockSpec((1, HW, Co), lambda d: (d, 0, 0)),
            pl.BlockSpec((1, HW, 1), lambda d: (d, 0, 0)),
        ],
        out_shape=[
            jax.ShapeDtypeStruct((Do, HW, Co), F32),
            jax.ShapeDtypeStruct((Do, HW, 1), F32),
        ],
        scratch_shapes=[pltpu.VMEM((Ho, Wo, 27 * C), F32)],
        compiler_params=pltpu.CompilerParams(
            vmem_limit_bytes=100 * 1024 * 1024),
        name=name,
    )


def _make_subm(Do, Ho, Wo, C, Co, name):
    HW = Ho * Wo

    def body(x0, x1, x2, m_ref, w_ref, b_ref, y_ref, col_ref):
        xs = (x0, x1, x2)
        for dz in range(3):
            for dy in range(3):
                for dx in range(3):
                    k = (dz * 3 + dy) * 3 + dx
                    col_ref[:, :, k * C:(k + 1) * C] = (
                        xs[dz][0, dy:dy + Ho, dx:dx + Wo, :])
        colr = col_ref[...].reshape(HW, 27 * C)
        ch = min(HW, 512)
        for i in range(HW // ch):
            lo, hi = i * ch, (i + 1) * ch
            acc = jnp.dot(colr[lo:hi], w_ref[...], preferred_element_type=F32,
                          precision=lax.Precision.HIGHEST)
            y_ref[0, lo:hi] = (jnp.maximum(acc + b_ref[...], 0.0)
                               * m_ref[0, lo:hi])

    x_specs = [pl.BlockSpec((1, Ho + 2, Wo + 2, C),
                            (lambda d, dz=dz: (d + dz, 0, 0, 0)))
               for dz in range(3)]
    return pl.pallas_call(
        body,
        grid=(Do,),
        in_specs=x_specs + [
            pl.BlockSpec((1, HW, 1), lambda d: (d, 0, 0)),
            pl.BlockSpec((27 * C, Co), lambda d: (0, 0)),
            pl.BlockSpec((1, Co), lambda d: (0, 0)),
        ],
        out_specs=pl.BlockSpec((1, HW, Co), lambda d: (d, 0, 0)),
        out_shape=jax.ShapeDtypeStruct((Do, HW, Co), F32),
        scratch_shapes=[pltpu.VMEM((Ho, Wo, 27 * C), F32)],
        compiler_params=pltpu.CompilerParams(
            vmem_limit_bytes=100 * 1024 * 1024),
        name=name,
    )


def _co_body(x0, x1, x2, m0, m1, m2, w_ref, b_ref, o_ref, col_ref):
    xs = (x0, x1, x2)
    ms = (m0, m1, m2)
    msum = jnp.zeros((256, 1), F32)
    for dz in range(3):
        col_ref[:, dz * 64:(dz + 1) * 64] = xs[dz][0]
        msum = msum + ms[dz][0]
    acc = jnp.dot(col_ref[...], w_ref[...], preferred_element_type=F32,
                  precision=lax.Precision.HIGHEST)
    nm = (msum > 0).astype(F32)
    o_ref[0] = jnp.maximum(acc + b_ref[...], 0.0) * nm


def _conv_out(x, m, w, b):
    x_specs = [pl.BlockSpec((1, 256, 64), (lambda d, dz=dz: (2 * d + dz, 0, 0)))
               for dz in range(3)]
    m_specs = [pl.BlockSpec((1, 256, 1), (lambda d, dz=dz: (2 * d + dz, 0, 0)))
               for dz in range(3)]
    return pl.pallas_call(
        _co_body,
        grid=(2,),
        in_specs=x_specs + m_specs + [
            pl.BlockSpec((192, 128), lambda d: (0, 0)),
            pl.BlockSpec((1, 128), lambda d: (0, 0)),
        ],
        out_specs=pl.BlockSpec((1, 256, 128), lambda d: (d, 0, 0)),
        out_shape=jax.ShapeDtypeStruct((2, 256, 128), F32),
        scratch_shapes=[pltpu.VMEM((256, 192), F32)],
        name="tc_conv_out",
    )(x, x, x, m, m, m, w, b)


def _fold_w(params, i):
    w = params['w%d' % i]
    scale = params['g%d' % i] / jnp.sqrt(1.0 + 1e-3)
    co, ci = w.shape[0], w.shape[1]
    wr = jnp.transpose(w, (2, 3, 4, 1, 0)).reshape(-1, co) * scale[None, :]
    return wr.astype(F32), params['b%d' % i].reshape(1, co).astype(F32)


def _parity(xp):
    d, h2, w2, c = xp.shape
    return (xp.reshape(d, h2 // 2, 2, w2 // 2, 2, c)
            .transpose(0, 1, 3, 2, 4, 5)
            .reshape(d, h2 // 2, w2 // 2, 4 * c))


def _pad3(x):
    return jnp.pad(x, ((1, 1), (1, 1), (1, 1), (0, 0)))


def kernel(voxel_features, voxel_coords, params):
    z = voxel_coords[:, 1] + 1
    y = voxel_coords[:, 2] + 1
    x = voxel_coords[:, 3] + 1

    ids = jnp.arange(NSITES, dtype=F32)[:, None]
    val8 = jnp.concatenate(
        [voxel_features.astype(F32), ids, jnp.zeros((NSITES, 3), F32)], axis=1)
    f0 = ((z * HP + y) * WP + x).astype(I32)
    order = jnp.argsort(f0, stable=True)
    f0s = f0[order]
    val8s = val8[order]
    keep = jnp.concatenate([f0s[:-1] != f0s[1:], jnp.ones((1,), bool)])
    idx_sc = jnp.where(keep, f0s, NPAD0)
    idx_scp = jnp.concatenate([idx_sc, jnp.full((NS - NSITES,), NPAD0, I32)])
    val8sp = jnp.concatenate([val8s, jnp.zeros((NS - NSITES, 8), F32)])
    zid = (NS + jnp.arange(NPAD0 + 8, dtype=I32) % ZPOOL).astype(F32)
    tref = jax.new_ref(jnp.zeros((NPAD0 + 8, 8), F32).at[:, 4].set(zid))
    _sc_scatter(8)(val8sp, idx_scp, tref)
    tab = tref[...]
    safe = jnp.full((NS - NSITES,), (1 * HP + 1) * WP + 1, I32)
    f0p = jnp.concatenate([f0, safe])
    offs = jnp.array([dz * HP * WP + dy * WP + dx
                      for dz in (-1, 0, 1) for dy in (-1, 0, 1)
                      for dx in (-1, 0, 1)], I32)
    idx27 = f0p[:, None] + offs[None, :]

    w0, b0 = _fold_w(params, 0)
    w0aug = jnp.zeros((27, 8, 16), F32).at[:, :4, :].set(
        w0.reshape(27, 4, 16)).reshape(216, 16)

    g0 = _gather27(tab, idx27, 8, "sc_gather_l0")
    y0 = _tc_l0(g0, w0aug, b0)
    nbr = lax.slice(g0, (0, 4), (NS, 216), (1, 8)).astype(I32)
    y0t = jnp.concatenate([y0, jnp.zeros((ZPOOL, 16), F32)], axis=0)

    w1, b1 = _fold_w(params, 1)
    g1 = _gather27(y0t, nbr, 16, "sc_gather_l1")
    y1 = _tc_l1(g1, w1, b1)

    p = (y & 1) * 2 + (x & 1)
    f2 = ((((z * 65 + (y >> 1)) * 65 + (x >> 1)) * 4) + p).astype(I32)
    f2p = jnp.concatenate([f2, jnp.full((NS - NSITES,), NG2, I32)])
    grid2 = _scatter16(y1, f2p, NG2 + 8)[:NG2].reshape(43, 65, 65, 64)

    md = (tab[:NPAD0, 4:5] < float(NS)).astype(F32).reshape(DP, HP, WP, 1)
    xm2 = jnp.concatenate([grid2, _parity(md)], axis=-1)

    w2, b2 = _fold_w(params, 2)
    y2, nm2 = _make_down(21, 64, 64, 65, 65, 16, 32, "tc_conv2")(
        xm2, xm2, xm2, w2, b2)

    w3, b3 = _fold_w(params, 3)
    x3 = _pad3(y2.reshape(21, 64, 64, 32))
    y3 = _make_subm(21, 64, 64, 32, 32, "tc_subm3")(x3, x3, x3, nm2, w3, b3)

    w4, b4 = _fold_w(params, 4)
    x4 = _pad3(y3.reshape(21, 64, 64, 32))
    y4 = _make_subm(21, 64, 64, 32, 32, "tc_subm4")(x4, x4, x4, nm2, w4, b4)

    w5, b5 = _fold_w(params, 5)
    xm3 = jnp.concatenate([_parity(_pad3(y4.reshape(21, 64, 64, 32))),
                           _parity(_pad3(nm2.reshape(21, 64, 64, 1)))],
                          axis=-1)
    y5, nm3 = _make_down(11, 32, 32, 33, 33, 32, 64, "tc_conv3")(
        xm3, xm3, xm3, w5, b5)

    w6, b6 = _fold_w(params, 6)
    x6 = _pad3(y5.reshape(11, 32, 32, 64))
    y6 = _make_subm(11, 32, 32, 64, 64, "tc_subm6")(x6, x6, x6, nm3, w6, b6)

    w7, b7 = _fold_w(params, 7)
    x7 = _pad3(y6.reshape(11, 32, 32, 64))
    y7 = _make_subm(11, 32, 32, 64, 64, "tc_subm7")(x7, x7, x7, nm3, w7, b7)

    w8, b8 = _fold_w(params, 8)
    padhw = ((0, 0), (1, 1), (1, 1), (0, 0))
    xm4 = jnp.concatenate(
        [_parity(jnp.pad(y7.reshape(11, 32, 32, 64), padhw)),
         _parity(jnp.pad(nm3.reshape(11, 32, 32, 1), padhw))],
        axis=-1)
    y8, nm4 = _make_down(5, 16, 16, 17, 17, 64, 64, "tc_conv4")(
        xm4, xm4, xm4, w8, b8)

    w9, b9 = _fold_w(params, 9)
    x9 = _pad3(y8.reshape(5, 16, 16, 64))
    y9 = _make_subm(5, 16, 16, 64, 64, "tc_subm9")(x9, x9, x9, nm4, w9, b9)

    w10, b10 = _fold_w(params, 10)
    x10 = _pad3(y9.reshape(5, 16, 16, 64))
    y10 = _make_subm(5, 16, 16, 64, 64, "tc_subm10")(x10, x10, x10, nm4,
                                                     w10, b10)

    w11, b11 = _fold_w(params, 11)
    out = _conv_out(y10, nm4, w11, b11)

    return out.reshape(2, 16, 16, 128).transpose(3, 0, 1, 2)[None]
